# Initial kernel scaffold; baseline (speedup 1.0000x reference)
import jax
import jax.numpy as jnp
from jax.experimental import pallas as pl


def kernel(user_id, item_id, edge_index, user_emb, item_emb):
    raise NotImplementedError("write your pallas kernel here")


# speedup vs baseline: 13.3802x; 13.3802x over previous
import functools

import jax
import jax.numpy as jnp
from jax import lax
from jax.experimental import pallas as pl
from jax.experimental.pallas import tpu as pltpu
from jax.experimental.pallas import tpu_sc as plsc

NU = 25000
NI = 25000
NN = NU + NI
E = 800000
D = 64
HD = 32
B = 4096

NC = 2
NS = 16
L = 16

EPT = E // NS
SCHUNK = 1024
NFULL = EPT // SCHUNK
TAIL = EPT - NFULL * SCHUNK

ACC_R = 50048
TRASH = NN
DRAIN = ACC_R // NS
HISTN = 25088
HSLC = HISTN // NS

_MESH = plsc.VectorSubcoreMesh(core_axis_name="c", subcore_axis_name="s")


def _vec_zero(ref, nvec):
    zero = jnp.zeros((L,), jnp.float32)
    if ref.shape == (nvec * L,):
        def body(i, c):
            ref[pl.ds(pl.multiple_of(i * L, L), L)] = zero
            return c
        lax.fori_loop(0, nvec, body, 0)
    else:
        rows, cols = ref.shape
        per = cols // L
        def body(i, c):
            for q in range(per):
                ref[i, pl.ds(q * L, L)] = zero
            return c
        lax.fori_loop(0, rows, body, 0)


def _deg_body(src_hbm, deg_hbm, ebuf, hist, rbuf, tbuf, shared, sem):
    c = lax.axis_index("c")
    s = lax.axis_index("s")
    base_node = c * NU

    _vec_zero(hist, HISTN // L)

    ones = jnp.full((L,), 1.0, jnp.float32)
    iota = lax.broadcasted_iota(jnp.int32, (L,), 0)
    tile_e = s * EPT

    def chunk(g, carry):
        ebase = tile_e + g * SCHUNK

        @pl.when(g < NFULL)
        def _():
            pltpu.sync_copy(src_hbm.at[pl.ds(ebase, SCHUNK)], ebuf)

        @pl.when(g == NFULL)
        def _():
            pltpu.sync_copy(src_hbm.at[pl.ds(ebase, TAIL)], ebuf.at[pl.ds(0, TAIL)])

        def inner(j, cc):
            v = ebuf[pl.ds(pl.multiple_of(j * L, L), L)]
            local = v - base_node
            lane = g * SCHUNK + j * L + iota
            valid = (local >= 0) & (local < NU) & (lane < EPT)
            idx = jnp.clip(local, 0, NU)
            plsc.addupdate_scatter(hist, [idx], ones, mask=valid)
            return cc
        lax.fori_loop(0, SCHUNK // L, inner, 0)
        return carry

    lax.fori_loop(0, NFULL + 1, chunk, 0)

    pltpu.sync_copy(hist, shared.at[pl.ds(s * HISTN, HISTN)])
    plsc.subcore_barrier()

    off = s * HSLC
    _vec_zero(rbuf, HSLC // L)

    def red(t, carry):
        pltpu.sync_copy(shared.at[pl.ds(t * HISTN + off, HSLC)], tbuf)

        def add16(j, cc):
            k = pl.ds(pl.multiple_of(j * L, L), L)
            rbuf[k] = rbuf[k] + tbuf[k]
            return cc
        lax.fori_loop(0, HSLC // L, add16, 0)
        return carry

    lax.fori_loop(0, NS, red, 0)

    gbase = base_node + off

    @pl.when(s < NS - 1)
    def _():
        pltpu.sync_copy(rbuf, deg_hbm.at[pl.ds(gbase, HSLC)])

    @pl.when(s == NS - 1)
    def _():
        last = NU - (NS - 1) * HSLC
        pltpu.sync_copy(rbuf.at[pl.ds(0, last)], deg_hbm.at[pl.ds(gbase, last)])


def _k_deg(src):
    f = pl.kernel(
        _deg_body,
        out_type=jax.ShapeDtypeStruct((NN,), jnp.float32),
        mesh=_MESH,
        compiler_params=pltpu.CompilerParams(needs_layout_passes=False, use_tc_tiling_on_sc=False),
        scratch_types=[
            pltpu.VMEM((SCHUNK,), jnp.int32),
            pltpu.VMEM((HISTN,), jnp.float32),
            pltpu.VMEM((HSLC,), jnp.float32),
            pltpu.VMEM((HSLC,), jnp.float32),
            pltpu.VMEM_SHARED((NS * HISTN,), jnp.float32),
            pltpu.SemaphoreType.DMA,
        ],
    )
    return f(src)


def _scale_body(deg_ref, ego_ref, g_ref, dinv_ref):
    d = deg_ref[...]
    dv = jnp.where(d > 0.0, lax.rsqrt(d), 0.0)
    e = ego_ref[...]
    g_ref[0] = dv * e[:, :HD]
    g_ref[1] = dv * e[:, HD:]
    dinv_ref[...] = dv


def _k_scale(deg2d, ego):
    rows = 1000
    grid = (NN // rows,)
    return pl.pallas_call(
        _scale_body,
        grid=grid,
        in_specs=[
            pl.BlockSpec((rows, 1), lambda i: (i, 0)),
            pl.BlockSpec((rows, D), lambda i: (i, 0)),
        ],
        out_specs=[
            pl.BlockSpec((NC, rows, HD), lambda i: (0, i, 0)),
            pl.BlockSpec((rows, 1), lambda i: (i, 0)),
        ],
        out_shape=[
            jax.ShapeDtypeStruct((NC, NN, HD), jnp.float32),
            jax.ShapeDtypeStruct((NN, 1), jnp.float32),
        ],
    )(deg2d, ego)


def _spmm_body(src_hbm, dst_hbm, g_hbm, a_hbm,
               sbuf, dbuf, sidx, didx, rows, zbuf, acc, sem):
    c = lax.axis_index("c")
    s = lax.axis_index("s")

    _vec_zero(zbuf, (DRAIN // 8) * HD // L)
    for q in range(8):
        pltpu.sync_copy(zbuf, acc.at[pl.ds(s * DRAIN + q * (DRAIN // 8), DRAIN // 8), :])
    plsc.subcore_barrier()

    goff = c * NN
    tile_e = s * EPT
    trash = jnp.full((L,), TRASH, jnp.int32)
    zero_i = jnp.zeros((L,), jnp.int32)

    def chunk(g, carry):
        ebase = tile_e + g * SCHUNK

        @pl.when(g < NFULL)
        def _():
            pltpu.sync_copy(src_hbm.at[pl.ds(ebase, SCHUNK)], sbuf)
            pltpu.sync_copy(dst_hbm.at[pl.ds(ebase, SCHUNK)], dbuf)

        @pl.when(g == NFULL)
        def _():
            for j in range(TAIL // L, SCHUNK // L):
                sbuf[pl.ds(j * L, L)] = trash
                dbuf[pl.ds(j * L, L)] = zero_i
            pltpu.sync_copy(src_hbm.at[pl.ds(ebase, TAIL)], sbuf.at[pl.ds(0, TAIL)])
            pltpu.sync_copy(dst_hbm.at[pl.ds(ebase, TAIL)], dbuf.at[pl.ds(0, TAIL)])

        def mv(r, cc):
            for q in range(8):
                k = pl.ds(pl.multiple_of(r * 128 + q * L, L), L)
                sidx[r, pl.ds(q * L, L)] = sbuf[k]
                didx[r, pl.ds(q * L, L)] = dbuf[k] + goff
            return cc
        lax.fori_loop(0, 8, mv, 0)

        for j in range(8):
            pltpu.async_copy(g_hbm.at[didx.at[j]], rows, sem).wait()
            pltpu.sync_copy(rows, acc.at[sidx.at[j]], add=True)
        return carry

    lax.fori_loop(0, NFULL + 1, chunk, 0)
    plsc.subcore_barrier()

    rbase = s * DRAIN

    @pl.when(s < NS - 1)
    def _():
        pltpu.sync_copy(acc.at[pl.ds(rbase, DRAIN), :],
                        a_hbm.at[pl.ds(c * NN + rbase, DRAIN), :])

    @pl.when(s == NS - 1)
    def _():
        last = NN - (NS - 1) * DRAIN
        pltpu.sync_copy(acc.at[pl.ds(rbase, last), :],
                        a_hbm.at[pl.ds(c * NN + rbase, last), :])


def _k_spmm(src, dst, g_flat):
    f = pl.kernel(
        _spmm_body,
        out_type=jax.ShapeDtypeStruct((NC * NN, HD), jnp.float32),
        mesh=_MESH,
        compiler_params=pltpu.CompilerParams(needs_layout_passes=False, use_tc_tiling_on_sc=False),
        scratch_types=[
            pltpu.VMEM((SCHUNK,), jnp.int32),
            pltpu.VMEM((SCHUNK,), jnp.int32),
            pltpu.VMEM((8, 128), jnp.int32),
            pltpu.VMEM((8, 128), jnp.int32),
            pltpu.VMEM((128, HD), jnp.float32),
            pltpu.VMEM((DRAIN // 8, HD), jnp.float32),
            pltpu.VMEM_SHARED((ACC_R, HD), jnp.float32),
            pltpu.SemaphoreType.DMA,
        ],
    )
    return f(src, dst, g_flat)


def _gscale_body(a_ref, dinv_ref, g_ref):
    dv = dinv_ref[...]
    g_ref[...] = a_ref[...] * (dv * dv)


def _k_gscale(a_flat, dinv_cat):
    rows = 1000
    grid = (NC * NN // rows,)
    return pl.pallas_call(
        _gscale_body,
        grid=grid,
        in_specs=[
            pl.BlockSpec((rows, HD), lambda i: (i, 0)),
            pl.BlockSpec((rows, 1), lambda i: (i, 0)),
        ],
        out_specs=pl.BlockSpec((rows, HD), lambda i: (i, 0)),
        out_shape=jax.ShapeDtypeStruct((NC * NN, HD), jnp.float32),
    )(a_flat, dinv_cat)


def _pick_body(uid_hbm, iid_hbm, uemb_hbm, iemb_hbm, a0_hbm, a1_hbm, dinv_hbm,
               uraw_hbm, iraw_hbm,
               idb, idbo, erows, a0h0, a0h1, a1h0, a1h1, dvb, obuf, sem):
    c = lax.axis_index("c")
    s = lax.axis_index("s")
    wid = c * NS + s
    nrow = B // (NC * NS)
    rbase = wid * nrow
    third = jnp.float32(1.0 / 3.0)

    for phase in range(2):
        id_hbm = uid_hbm if phase == 0 else iid_hbm
        emb_hbm = uemb_hbm if phase == 0 else iemb_hbm
        out_hbm = uraw_hbm if phase == 0 else iraw_hbm
        node_off = 0 if phase == 0 else NU

        pltpu.sync_copy(id_hbm.at[pl.ds(rbase, nrow)], idb)
        pltpu.async_copy(emb_hbm.at[idb], erows, sem).wait()

        def shift(off_const, dst_ref):
            off = jnp.full((L,), off_const, jnp.int32)
            def body(j, cc):
                k = pl.ds(pl.multiple_of(j * L, L), L)
                dst_ref[k] = idb[k] + off
                return cc
            lax.fori_loop(0, nrow // L, body, 0)

        shift(node_off, idbo)
        pltpu.async_copy(a0_hbm.at[idbo], a0h0, sem).wait()
        pltpu.async_copy(a1_hbm.at[idbo], a1h0, sem).wait()
        pltpu.async_copy(dinv_hbm.at[idbo], dvb, sem).wait()
        shift(node_off + NN, idbo)
        pltpu.async_copy(a0_hbm.at[idbo], a0h1, sem).wait()
        pltpu.async_copy(a1_hbm.at[idbo], a1h1, sem).wait()

        def row(r, cc):
            dvr = plsc.load_gather(dvb, [jnp.full((L,), r, jnp.int32)])
            for q in range(4):
                a0t = a0h0 if q < 2 else a0h1
                a1t = a1h0 if q < 2 else a1h1
                kk = pl.ds((q % 2) * L, L)
                u = (erows[r, pl.ds(q * L, L)]
                     + dvr * (a0t[r, kk] + a1t[r, kk])) * third
                obuf[r, pl.ds(q * L, L)] = u
            return cc
        lax.fori_loop(0, nrow, row, 0)

        pltpu.sync_copy(obuf, out_hbm.at[pl.ds(rbase, nrow)])


def _k_pick(uid, iid, uemb, iemb, a0_flat, a1_flat, dinv1d):
    nrow = B // (NC * NS)
    f = pl.kernel(
        _pick_body,
        out_type=[
            jax.ShapeDtypeStruct((B, D), jnp.float32),
            jax.ShapeDtypeStruct((B, D), jnp.float32),
        ],
        mesh=_MESH,
        compiler_params=pltpu.CompilerParams(needs_layout_passes=False, use_tc_tiling_on_sc=False),
        scratch_types=[
            pltpu.VMEM((nrow,), jnp.int32),
            pltpu.VMEM((nrow,), jnp.int32),
            pltpu.VMEM((nrow, D), jnp.float32),
            pltpu.VMEM((nrow, HD), jnp.float32),
            pltpu.VMEM((nrow, HD), jnp.float32),
            pltpu.VMEM((nrow, HD), jnp.float32),
            pltpu.VMEM((nrow, HD), jnp.float32),
            pltpu.VMEM((nrow,), jnp.float32),
            pltpu.VMEM((nrow, D), jnp.float32),
            pltpu.SemaphoreType.DMA,
        ],
    )
    return f(uid, iid, uemb, iemb, a0_flat, a1_flat, dinv1d)


def _norm_body(u_ref, i_ref, un_ref, in_ref):
    for src, dstref in ((u_ref, un_ref), (i_ref, in_ref)):
        x = src[...]
        n = jnp.sqrt(jnp.sum(x * x, axis=1, keepdims=True))
        dstref[...] = x / jnp.maximum(n, 1e-12)


def _k_norm(u_raw, i_raw):
    rows = 512
    grid = (B // rows,)
    spec = pl.BlockSpec((rows, D), lambda i: (i, 0))
    return pl.pallas_call(
        _norm_body,
        grid=grid,
        in_specs=[spec, spec],
        out_specs=[spec, spec],
        out_shape=[
            jax.ShapeDtypeStruct((B, D), jnp.float32),
            jax.ShapeDtypeStruct((B, D), jnp.float32),
        ],
    )(u_raw, i_raw)


def kernel(user_id, item_id, edge_index, user_emb, item_emb):
    src = edge_index[0].astype(jnp.int32)
    dst = edge_index[1].astype(jnp.int32)
    user_id = user_id.astype(jnp.int32)
    item_id = item_id.astype(jnp.int32)
    ego = jnp.concatenate([user_emb, item_emb], axis=0)

    deg = _k_deg(src)
    g0_2, dinv2d = _k_scale(deg.reshape(NN, 1), ego)
    g0 = g0_2.reshape(NC * NN, HD)
    a0 = _k_spmm(src, dst, g0)
    dinv_cat = jnp.concatenate([dinv2d, dinv2d], axis=0)
    g1 = _k_gscale(a0, dinv_cat)
    a1 = _k_spmm(src, dst, g1)
    u_raw, i_raw = _k_pick(user_id, item_id, user_emb, item_emb, a0, a1,
                           dinv2d.reshape(NN))
    return _k_norm(u_raw, i_raw)
